# baseline (speedup 1.0000x reference)
import jax
import jax.numpy as jnp
from jax.experimental import pallas as pl
from jax.experimental.pallas import tpu as pltpu

_ROWS = 1000
_COLS = 16384
_BLOCK_COLS = 2048
_GRID = _COLS // _BLOCK_COLS


def _mse_body(yt_ref, yp_ref, out_ref, acc_ref):
    i = pl.program_id(0)

    @pl.when(i == 0)
    def _init():
        acc_ref[0] = 0.0
        acc_ref[1] = 0.0

    yt = yt_ref[...]
    yp = yp_ref[...]
    mask = (yt != 0.0) & (yp != 0.0)
    d = yt - yp
    sq = jnp.where(mask, d * d, 0.0)
    acc_ref[0] += jnp.sum(sq)
    acc_ref[1] += jnp.sum(mask.astype(jnp.float32))

    @pl.when(i == _GRID - 1)
    def _fin():
        out_ref[0, 0] = acc_ref[0] / acc_ref[1]


def kernel(y_true, y_pred):
    out = pl.pallas_call(
        _mse_body,
        grid=(_GRID,),
        in_specs=[
            pl.BlockSpec((_ROWS, _BLOCK_COLS), lambda i: (0, i)),
            pl.BlockSpec((_ROWS, _BLOCK_COLS), lambda i: (0, i)),
        ],
        out_specs=pl.BlockSpec(
            (1, 1), lambda i: (0, 0), memory_space=pltpu.SMEM
        ),
        out_shape=jax.ShapeDtypeStruct((1, 1), jnp.float32),
        scratch_shapes=[pltpu.SMEM((2,), jnp.float32)],
    )(y_true.T, y_pred.T)
    return out[0, 0]
